# baseline (speedup 1.0000x reference)
import functools

import jax
import jax.numpy as jnp
from jax import lax
from jax.experimental import pallas as pl
from jax.experimental.pallas import tpu as pltpu
from jax.experimental.pallas import tpu_sc as plsc

PAD_ID = 999999
D = 64
NC = 2
NS = 16
NW = NC * NS

CHUNK = 40
NBUF = 2


def _prep_body(idx_ref, idx2_ref, off_ref, mask_ref):
    idx = idx_ref[...]
    idx2_ref[...] = idx >> 1
    off_ref[...] = (idx & 1) * D
    mask_ref[...] = idx == PAD_ID


def _gather_body(n_chunks, idx2_hbm, off_hbm, t2_hbm, out_hbm,
                 idx2_v, off_v, rows, comp, gsems, osems):
    wid = lax.axis_index("s") * NC + lax.axis_index("c")
    sent0 = wid * 128
    cps = 200 // CHUNK

    pltpu.sync_copy(idx2_hbm.at[wid], idx2_v)
    pltpu.sync_copy(off_hbm.at[wid], off_v)

    def gather(g, j):
        return pltpu.async_copy(
            t2_hbm.at[idx2_v.at[pl.ds(g * CHUNK, CHUNK)]], rows[j], gsems[j])

    def out_slice(g):
        return out_hbm.at[sent0 + g // cps, pl.ds((g % cps) * CHUNK, CHUNK)]

    def put(g, j):
        return pltpu.async_copy(comp[j], out_slice(g), osems[j])

    def wait_gather(g, j):
        pltpu.make_async_copy(
            t2_hbm.at[idx2_v.at[pl.ds(g * CHUNK, CHUNK)]], rows[j],
            gsems[j]).wait()

    def wait_put(g, j):
        pltpu.make_async_copy(comp[j], out_slice(g), osems[j]).wait()

    def compact(g, j):
        ovs = [off_v[pl.ds(g * CHUNK + r0, 16)] for r0 in (0, 16, 24)]
        for r in range(CHUNK):
            if r < 32:
                off = ovs[r // 16][r % 16]
            else:
                off = ovs[2][r - 24]
            take_hi = off > 0
            for k in range(D // 16):
                lo = rows[j][r, pl.ds(k * 16, 16)]
                hi = rows[j][r, pl.ds(D + k * 16, 16)]
                comp[j][r, pl.ds(k * 16, 16)] = jnp.where(take_hi, hi, lo)

    for j in range(NBUF):
        gather(j, j)

    @pl.loop(0, n_chunks - NBUF, step=NBUF)
    def _(g0):
        for j in range(NBUF):
            g = g0 + j
            wait_gather(g, j)
            compact(g, j)
            put(g, j)
            wait_put(g, j)
            gather(g + NBUF, j)

    for j in range(NBUF):
        g = n_chunks - NBUF + j
        wait_gather(g, j)
        compact(g, j)
        put(g, j)
    for j in range(NBUF):
        wait_put(n_chunks - NBUF + j, j)


@jax.jit
def kernel(word_indices, vocabulary):
    n_rows, seq = word_indices.shape
    b = n_rows * seq
    n_chunks = b // (NW * CHUNK)

    mesh = plsc.VectorSubcoreMesh(core_axis_name="c", subcore_axis_name="s")
    sc_params = pltpu.CompilerParams(use_tc_tiling_on_sc=True,
                                     needs_layout_passes=False)

    idx2, off, mask = pl.pallas_call(
        _prep_body,
        out_shape=(
            jax.ShapeDtypeStruct((n_rows, seq), jnp.int32),
            jax.ShapeDtypeStruct((n_rows, seq), jnp.int32),
            jax.ShapeDtypeStruct((n_rows, seq), jnp.bool_),
        ),
    )(word_indices)

    t2 = vocabulary.reshape(500000, 128)

    embedded = pl.kernel(
        functools.partial(_gather_body, n_chunks),
        out_type=jax.ShapeDtypeStruct((n_rows, seq, D), jnp.float32),
        mesh=mesh,
        scratch_types=[
            pltpu.VMEM((n_chunks * CHUNK,), jnp.int32),
            pltpu.VMEM((n_chunks * CHUNK,), jnp.int32),
            tuple(pltpu.VMEM((CHUNK, 128), jnp.float32) for _ in range(NBUF)),
            tuple(pltpu.VMEM((CHUNK, D), jnp.float32) for _ in range(NBUF)),
            tuple(pltpu.SemaphoreType.DMA for _ in range(NBUF)),
            tuple(pltpu.SemaphoreType.DMA for _ in range(NBUF)),
        ],
        compiler_params=sc_params,
    )(idx2.reshape(NW, n_chunks * CHUNK), off.reshape(NW, n_chunks * CHUNK),
      t2)

    return embedded, mask


# speedup vs baseline: 1.2075x; 1.2075x over previous
import functools

import jax
import jax.numpy as jnp
from jax import lax
from jax.experimental import pallas as pl
from jax.experimental.pallas import tpu as pltpu
from jax.experimental.pallas import tpu_sc as plsc

PAD_ID = 999999
D = 64

NC = 2
NS = 16
NW = NC * NS

NBUF = 4
NUM_CORES = 2


def _gather_body(n_chunks, chunk, idx_hbm, table_hbm, out_hbm,
                 idx_v, rows, gsems, osems):
    wid = lax.axis_index("s") * NUM_CORES + lax.axis_index("c")
    bpw = n_chunks * chunk
    base = wid * bpw

    pltpu.sync_copy(idx_hbm.at[wid], idx_v)

    def gather(g, j):
        return pltpu.async_copy(table_hbm.at[idx_v.at[g]], rows[j], gsems[j])

    def out_slice(g):
        return out_hbm.at[pl.ds(base + g * chunk, chunk)]

    def put(g, j):
        return pltpu.async_copy(rows[j], out_slice(g), osems[j])

    def wait_gather(g, j):
        pltpu.make_async_copy(table_hbm.at[idx_v.at[g]], rows[j],
                              gsems[j]).wait()

    def wait_put(g, j):
        pltpu.make_async_copy(rows[j], out_slice(g), osems[j]).wait()

    for j in range(NBUF):
        gather(j, j)

    @pl.loop(0, n_chunks - NBUF, step=NBUF)
    def _(g0):
        for j in range(NBUF):
            g = g0 + j
            wait_gather(g, j)
            put(g, j)
            wait_put(g, j)
            gather(g + NBUF, j)

    for j in range(NBUF):
        g = n_chunks - NBUF + j
        wait_gather(g, j)
        put(g, j)
    for j in range(NBUF):
        wait_put(n_chunks - NBUF + j, j)


def _mask_body(idx_ref, out_ref):
    out_ref[...] = idx_ref[...] == PAD_ID


@jax.jit
def kernel(word_indices, vocabulary):
    n_rows, seq = word_indices.shape
    b = n_rows * seq
    bpw = b // (NUM_CORES * NS)
    chunk = 128
    n_chunks = bpw // chunk

    idx_flat = word_indices.reshape(NUM_CORES * NS, n_chunks, chunk)

    mesh = plsc.VectorSubcoreMesh(core_axis_name="c", subcore_axis_name="s",
                                  num_cores=NUM_CORES)
    gathered = pl.kernel(
        functools.partial(_gather_body, n_chunks, chunk),
        out_type=jax.ShapeDtypeStruct((b, D), jnp.float32),
        mesh=mesh,
        scratch_types=[
            pltpu.VMEM((n_chunks, chunk), jnp.int32),
            tuple(pltpu.VMEM((chunk, D), jnp.float32) for _ in range(NBUF)),
            tuple(pltpu.SemaphoreType.DMA for _ in range(NBUF)),
            tuple(pltpu.SemaphoreType.DMA for _ in range(NBUF)),
        ],
        compiler_params=pltpu.CompilerParams(use_tc_tiling_on_sc=False),
    )(idx_flat, vocabulary)

    mask = pl.pallas_call(
        _mask_body,
        out_shape=jax.ShapeDtypeStruct((n_rows, seq), jnp.bool_),
    )(word_indices)

    return gathered.reshape(n_rows, seq, D), mask
